# baseline (speedup 1.0000x reference)
import numpy as np
import jax
import jax.numpy as jnp
from jax import lax
from jax.experimental import pallas as pl
from jax.experimental.pallas import tpu as pltpu
from jax.experimental.pallas import tpu_sc as plsc

_N_LEVELS = 16
_LOG2_T = 19
_T = 1 << _LOG2_T
_BASE_RES = 16
_SCALE = 1.26
_N = 1048576
_PRIME1 = np.int32(np.uint32(2654435761).view(np.int32))
_PRIME2 = np.int32(805459861)
_RES = [int(np.ceil(_BASE_RES * (_SCALE ** l))) for l in range(_N_LEVELS)]
_DENSE = [(r + 1) ** 3 <= _T for r in _RES]

_NC, _NS, _L = 2, 16, 16
_NW = _NC * _NS
_PTS_PER_TILE = _N // _NW
_P = 2048
_CHUNKS = _PTS_PER_TILE // _P
_STRIPS = _P // _L
_NIDX = 8 * _P


def _encode_body(xf, tab, encT, xbuf, idxb0, idxb1, wcb, rf0, rf1,
                 f0a, f1a, sem):
    wid = lax.axis_index("s") * _NC + lax.axis_index("c")
    tile_base = wid * _PTS_PER_TILE

    @pl.loop(0, _CHUNKS)
    def _chunk(ci):
        base = tile_base + ci * _P
        pltpu.sync_copy(xf.at[pl.ds(3 * base, 3 * _P)], xbuf)

        for l in range(_N_LEVELS):
            res = _RES[l]
            dense = _DENSE[l]
            fres = jnp.float32(res)

            @pl.loop(0, _STRIPS)
            def _pa(s):
                o = s * _L
                xi = lax.iota(jnp.int32, _L) * 3 + (3 * o)
                xs = (plsc.load_gather(xbuf, [xi]),
                      plsc.load_gather(xbuf, [xi + 1]),
                      plsc.load_gather(xbuf, [xi + 2]))
                pis = []
                wlo = []
                whi = []
                for d in range(3):
                    pos = xs[d] * fres
                    pi = pos.astype(jnp.int32)
                    fr = pos - pi.astype(jnp.float32)
                    w = fr * fr * (3.0 - 2.0 * fr)
                    pis.append(pi)
                    whi.append(w)
                    wlo.append(1.0 - w)
                for corner in range(8):
                    bits = [(corner >> d) & 1 for d in range(3)]
                    c0 = pis[0] + bits[0] if bits[0] else pis[0]
                    c1 = pis[1] + bits[1] if bits[1] else pis[1]
                    c2 = pis[2] + bits[2] if bits[2] else pis[2]
                    if dense:
                        idx = c0 + c1 * (res + 1) + c2 * ((res + 1) * (res + 1))
                    else:
                        idx = (c0 ^ (c1 * _PRIME1) ^ (c2 * _PRIME2)) & (_T - 1)
                    wc = ((whi[0] if bits[0] else wlo[0])
                          * (whi[1] if bits[1] else wlo[1])
                          * (whi[2] if bits[2] else wlo[2]))
                    g2 = (idx + (l * _T)) * 2
                    idxb0[pl.ds(corner * _P + o, _L)] = g2
                    idxb1[pl.ds(corner * _P + o, _L)] = g2 + 1
                    wcb[pl.ds(corner * _P + o, _L)] = wc

            cp0 = pltpu.async_copy(tab.at[idxb0], rf0, sem)
            cp1 = pltpu.async_copy(tab.at[idxb1], rf1, sem)
            cp0.wait()
            cp1.wait()

            @pl.loop(0, _STRIPS)
            def _pc(s):
                o = s * _L
                acc0 = jnp.zeros((_L,), jnp.float32)
                acc1 = jnp.zeros((_L,), jnp.float32)
                for corner in range(8):
                    wc = wcb[pl.ds(corner * _P + o, _L)]
                    f0 = rf0[pl.ds(corner * _P + o, _L)]
                    f1 = rf1[pl.ds(corner * _P + o, _L)]
                    acc0 = acc0 + f0 * wc
                    acc1 = acc1 + f1 * wc
                f0a[pl.ds(o, _L)] = acc0
                f1a[pl.ds(o, _L)] = acc1

            pltpu.sync_copy(f0a, encT.at[2 * l, pl.ds(base, _P)])
            pltpu.sync_copy(f1a, encT.at[2 * l + 1, pl.ds(base, _P)])


def _encode(xT, tab):
    mesh = plsc.VectorSubcoreMesh(core_axis_name="c", subcore_axis_name="s")
    return pl.kernel(
        _encode_body,
        out_type=jax.ShapeDtypeStruct((2 * _N_LEVELS, _N), jnp.float32),
        mesh=mesh,
        compiler_params=pltpu.CompilerParams(
            use_tc_tiling_on_sc=False, needs_layout_passes=False),
        scratch_types=[
            pltpu.VMEM((3 * _P,), jnp.float32),
            pltpu.VMEM((_NIDX,), jnp.int32),
            pltpu.VMEM((_NIDX,), jnp.int32),
            pltpu.VMEM((_NIDX,), jnp.float32),
            pltpu.VMEM((_NIDX,), jnp.float32),
            pltpu.VMEM((_NIDX,), jnp.float32),
            pltpu.VMEM((_P,), jnp.float32),
            pltpu.VMEM((_P,), jnp.float32),
            pltpu.SemaphoreType.DMA,
        ],
    )(xT, tab)


_MLP_B = 2048


def _mlp_body(enc_ref, w1_ref, w2_ref, out_ref):
    e = enc_ref[...]
    h = lax.dot_general(e, w1_ref[...], (((0,), (0,)), ((), ())),
                        preferred_element_type=jnp.float32)
    h = jnp.maximum(h, 0.0)
    o = jnp.dot(h, w2_ref[...], preferred_element_type=jnp.float32)
    out_ref[...] = jnp.clip(o[:, :3], 0.0, 1.0)


def _mlp(encT, W1, W2p):
    return pl.pallas_call(
        _mlp_body,
        grid=(_N // _MLP_B,),
        in_specs=[
            pl.BlockSpec((2 * _N_LEVELS, _MLP_B), lambda i: (0, i)),
            pl.BlockSpec((32, 64), lambda i: (0, 0)),
            pl.BlockSpec((64, 8), lambda i: (0, 0)),
        ],
        out_specs=pl.BlockSpec((_MLP_B, 3), lambda i: (i, 0)),
        out_shape=jax.ShapeDtypeStruct((_N, 3), jnp.float32),
    )(encT, W1, W2p)


def kernel(x, table, W1, W2):
    xf = x.reshape(_N * 3)
    tab = table.reshape(_N_LEVELS * _T * 2)
    encT = _encode(xf, tab)
    W2p = jnp.zeros((64, 8), jnp.float32).at[:, :3].set(W2)
    return _mlp(encT, W1, W2p)


# speedup vs baseline: 2.3248x; 2.3248x over previous
import numpy as np
import jax
import jax.numpy as jnp
from jax import lax
from jax.experimental import pallas as pl
from jax.experimental.pallas import tpu as pltpu
from jax.experimental.pallas import tpu_sc as plsc

_N_LEVELS = 16
_LOG2_T = 19
_T = 1 << _LOG2_T
_BASE_RES = 16
_SCALE = 1.26
_N = 1048576
_PRIME1 = np.int32(np.uint32(2654435761).view(np.int32))
_PRIME2 = np.int32(805459861)
_RES = [int(np.ceil(_BASE_RES * (_SCALE ** l))) for l in range(_N_LEVELS)]
_DENSE = [(r + 1) ** 3 <= _T for r in _RES]

_NC, _NS, _L = 2, 16, 16
_NW = _NC * _NS
_PTS_PER_TILE = _N // _NW
_P = 2048
_CHUNKS = _PTS_PER_TILE // _P
_STRIPS = _P // _L
_NIDX = 8 * _P


def _encode_body(xf, tab, encT, xbuf, idxb, wcb, rf, f0a, f1a, sem):
    wid = lax.axis_index("s") * _NC + lax.axis_index("c")
    tile_base = wid * _PTS_PER_TILE

    @pl.loop(0, _CHUNKS)
    def _chunk(ci):
        base = tile_base + ci * _P
        pltpu.sync_copy(xf.at[pl.ds(3 * base, 3 * _P)], xbuf)

        for l in range(_N_LEVELS):
            res = _RES[l]
            dense = _DENSE[l]
            fres = jnp.float32(res)

            @pl.loop(0, _STRIPS)
            def _pa(s):
                o = s * _L
                xi = lax.iota(jnp.int32, _L) * 3 + (3 * o)
                xs = (plsc.load_gather(xbuf, [xi]),
                      plsc.load_gather(xbuf, [xi + 1]),
                      plsc.load_gather(xbuf, [xi + 2]))
                pis = []
                wlo = []
                whi = []
                for d in range(3):
                    pos = xs[d] * fres
                    pi = pos.astype(jnp.int32)
                    fr = pos - pi.astype(jnp.float32)
                    w = fr * fr * (3.0 - 2.0 * fr)
                    pis.append(pi)
                    whi.append(w)
                    wlo.append(1.0 - w)
                for corner in range(8):
                    bits = [(corner >> d) & 1 for d in range(3)]
                    c0 = pis[0] + bits[0] if bits[0] else pis[0]
                    c1 = pis[1] + bits[1] if bits[1] else pis[1]
                    c2 = pis[2] + bits[2] if bits[2] else pis[2]
                    if dense:
                        idx = c0 + c1 * (res + 1) + c2 * ((res + 1) * (res + 1))
                    else:
                        idx = (c0 ^ (c1 * _PRIME1) ^ (c2 * _PRIME2)) & (_T - 1)
                    wc = ((whi[0] if bits[0] else wlo[0])
                          * (whi[1] if bits[1] else wlo[1])
                          * (whi[2] if bits[2] else wlo[2]))
                    idxb[pl.ds(corner * _P + o, _L)] = idx + (l * _T)
                    wcb[pl.ds(corner * _P + o, _L)] = wc

            pltpu.async_copy(tab.at[idxb], rf, sem).wait()

            @pl.loop(0, _STRIPS)
            def _pc(s):
                o = s * _L
                acc0 = jnp.zeros((_L,), jnp.float32)
                acc1 = jnp.zeros((_L,), jnp.float32)
                for corner in range(8):
                    wc = wcb[pl.ds(corner * _P + o, _L)]
                    r = rf[pl.ds(corner * _P + o, _L)]
                    ab = plsc.bitcast(r, jnp.bfloat16)
                    f0, f1 = plsc.unpack(ab, format=plsc.PackFormat.INTERLEAVED)
                    acc0 = acc0 + f0 * wc
                    acc1 = acc1 + f1 * wc
                f0a[pl.ds(o, _L)] = acc0
                f1a[pl.ds(o, _L)] = acc1

            pltpu.sync_copy(f0a, encT.at[2 * l, pl.ds(base, _P)])
            pltpu.sync_copy(f1a, encT.at[2 * l + 1, pl.ds(base, _P)])


def _encode(xT, tab):
    mesh = plsc.VectorSubcoreMesh(core_axis_name="c", subcore_axis_name="s")
    return pl.kernel(
        _encode_body,
        out_type=jax.ShapeDtypeStruct((2 * _N_LEVELS, _N), jnp.float32),
        mesh=mesh,
        compiler_params=pltpu.CompilerParams(
            use_tc_tiling_on_sc=False, needs_layout_passes=False),
        scratch_types=[
            pltpu.VMEM((3 * _P,), jnp.float32),
            pltpu.VMEM((_NIDX,), jnp.int32),
            pltpu.VMEM((_NIDX,), jnp.float32),
            pltpu.VMEM((_NIDX,), jnp.uint32),
            pltpu.VMEM((_P,), jnp.float32),
            pltpu.VMEM((_P,), jnp.float32),
            pltpu.SemaphoreType.DMA,
        ],
    )(xT, tab)


def _pack_body(tv_ref, out_ref):
    f0 = tv_ref[0, :, 0, :]
    f1 = tv_ref[0, :, 1, :]
    u0 = lax.bitcast_convert_type(f0.astype(jnp.bfloat16), jnp.uint16)
    u1 = lax.bitcast_convert_type(f1.astype(jnp.bfloat16), jnp.uint16)
    out_ref[0] = u0.astype(jnp.uint32) | (u1.astype(jnp.uint32) << 16)


def _pack_table(tv):
    out = pl.pallas_call(
        _pack_body,
        grid=(_N_LEVELS, 8),
        in_specs=[pl.BlockSpec((1, 512, 2, 128), lambda i, j: (i, j, 0, 0))],
        out_specs=pl.BlockSpec((1, 512, 128), lambda i, j: (i, j, 0)),
        out_shape=jax.ShapeDtypeStruct((_N_LEVELS, 4096, 128), jnp.uint32),
    )(tv)
    return out.reshape(_N_LEVELS * _T)


_MLP_B = 2048


def _mlp_body(enc_ref, w1_ref, w2_ref, out_ref):
    e = enc_ref[...]
    h = lax.dot_general(e, w1_ref[...], (((0,), (0,)), ((), ())),
                        preferred_element_type=jnp.float32)
    h = jnp.maximum(h, 0.0)
    o = jnp.dot(h, w2_ref[...], preferred_element_type=jnp.float32)
    out_ref[...] = jnp.clip(o[:, :3], 0.0, 1.0)


def _mlp(encT, W1, W2p):
    return pl.pallas_call(
        _mlp_body,
        grid=(_N // _MLP_B,),
        in_specs=[
            pl.BlockSpec((2 * _N_LEVELS, _MLP_B), lambda i: (0, i)),
            pl.BlockSpec((32, 64), lambda i: (0, 0)),
            pl.BlockSpec((64, 8), lambda i: (0, 0)),
        ],
        out_specs=pl.BlockSpec((_MLP_B, 3), lambda i: (i, 0)),
        out_shape=jax.ShapeDtypeStruct((_N, 3), jnp.float32),
    )(encT, W1, W2p)


def kernel(x, table, W1, W2):
    xf = x.reshape(_N * 3)
    tv = table.reshape(_N_LEVELS, _T // 128, 128, 2).transpose(0, 1, 3, 2)
    tab = _pack_table(tv)
    encT = _encode(xf, tab)
    W2p = jnp.zeros((64, 8), jnp.float32).at[:, :3].set(W2)
    return _mlp(encT, W1, W2p)


# speedup vs baseline: 2.9793x; 1.2815x over previous
import numpy as np
import jax
import jax.numpy as jnp
from jax import lax
from jax.experimental import pallas as pl
from jax.experimental.pallas import tpu as pltpu
from jax.experimental.pallas import tpu_sc as plsc

_N_LEVELS = 16
_LOG2_T = 19
_T = 1 << _LOG2_T
_BASE_RES = 16
_SCALE = 1.26
_N = 1048576
_PRIME1 = np.int32(np.uint32(2654435761).view(np.int32))
_PRIME2 = np.int32(805459861)
_RES = [int(np.ceil(_BASE_RES * (_SCALE ** l))) for l in range(_N_LEVELS)]
_DENSE = [(r + 1) ** 3 <= _T for r in _RES]

_NC, _NS, _L = 2, 16, 16
_NW = _NC * _NS
_PTS_PER_TILE = _N // _NW
_P = 2048
_CHUNKS = _PTS_PER_TILE // _P
_STRIPS = _P // _L
_NIDX = 8 * _P


def _encode_body(xf, tab, encT, xbuf, idxb, wcb, rf, f0a, f1a, sem):
    wid = lax.axis_index("s") * _NC + lax.axis_index("c")
    tile_base = wid * _PTS_PER_TILE

    @pl.loop(0, _CHUNKS)
    def _chunk(ci):
        base = tile_base + ci * _P
        pltpu.sync_copy(xf.at[pl.ds(3 * base, 3 * _P)], xbuf)

        for l in range(_N_LEVELS):
            res = _RES[l]
            dense = _DENSE[l]
            fres = jnp.float32(res)

            @pl.loop(0, _STRIPS)
            def _pa(s):
                o = s * _L
                xi = lax.iota(jnp.int32, _L) * 3 + (3 * o)
                xs = (plsc.load_gather(xbuf, [xi]),
                      plsc.load_gather(xbuf, [xi + 1]),
                      plsc.load_gather(xbuf, [xi + 2]))
                pis = []
                wlo = []
                whi = []
                for d in range(3):
                    pos = xs[d] * fres
                    pi = pos.astype(jnp.int32)
                    fr = pos - pi.astype(jnp.float32)
                    w = fr * fr * (3.0 - 2.0 * fr)
                    pis.append(pi)
                    whi.append(w)
                    wlo.append(1.0 - w)
                for corner in range(8):
                    bits = [(corner >> d) & 1 for d in range(3)]
                    c0 = pis[0] + bits[0] if bits[0] else pis[0]
                    c1 = pis[1] + bits[1] if bits[1] else pis[1]
                    c2 = pis[2] + bits[2] if bits[2] else pis[2]
                    if dense:
                        idx = c0 + c1 * (res + 1) + c2 * ((res + 1) * (res + 1))
                    else:
                        idx = (c0 ^ (c1 * _PRIME1) ^ (c2 * _PRIME2)) & (_T - 1)
                    wc = ((whi[0] if bits[0] else wlo[0])
                          * (whi[1] if bits[1] else wlo[1])
                          * (whi[2] if bits[2] else wlo[2]))
                    idxb[pl.ds(corner * _P + o, _L)] = idx + (l * _T)
                    wcb[pl.ds(corner * _P + o, _L)] = wc

            pltpu.async_copy(tab.at[idxb], rf, sem).wait()

            @pl.loop(0, _P // 128)
            def _pcg(cg):
                @pl.loop(0, 128 // _L)
                def _pc(ss):
                    o = cg * 128 + ss * _L
                    acc0 = jnp.zeros((_L,), jnp.float32)
                    acc1 = jnp.zeros((_L,), jnp.float32)
                    for corner in range(8):
                        wc = wcb[pl.ds(corner * _P + o, _L)]
                        r = rf[pl.ds(corner * _P + o, _L)]
                        ab = plsc.bitcast(r, jnp.bfloat16)
                        f0, f1 = plsc.unpack(ab, format=plsc.PackFormat.INTERLEAVED)
                        acc0 = acc0 + f0 * wc
                        acc1 = acc1 + f1 * wc
                    f0a[cg, pl.ds(ss * _L, _L)] = acc0
                    f1a[cg, pl.ds(ss * _L, _L)] = acc1

            cbase = wid * (_PTS_PER_TILE // 128) + ci * (_P // 128)
            pltpu.sync_copy(f0a, encT.at[2 * l, pl.ds(cbase, _P // 128), :])
            pltpu.sync_copy(f1a, encT.at[2 * l + 1, pl.ds(cbase, _P // 128), :])


def _encode(xT, tab):
    mesh = plsc.VectorSubcoreMesh(core_axis_name="c", subcore_axis_name="s")
    return pl.kernel(
        _encode_body,
        out_type=jax.ShapeDtypeStruct((2 * _N_LEVELS, _N // 128, 128),
                                      jnp.float32),
        mesh=mesh,
        compiler_params=pltpu.CompilerParams(
            use_tc_tiling_on_sc=False, needs_layout_passes=False),
        scratch_types=[
            pltpu.VMEM((3 * _P,), jnp.float32),
            pltpu.VMEM((_NIDX,), jnp.int32),
            pltpu.VMEM((_NIDX,), jnp.float32),
            pltpu.VMEM((_NIDX,), jnp.uint32),
            pltpu.VMEM((_P // 128, 128), jnp.float32),
            pltpu.VMEM((_P // 128, 128), jnp.float32),
            pltpu.SemaphoreType.DMA,
        ],
    )(xT, tab)


def _pack_body(tv_ref, out_ref):
    f0 = tv_ref[0, :, 0, :]
    f1 = tv_ref[0, :, 1, :]
    u0 = lax.bitcast_convert_type(f0.astype(jnp.bfloat16), jnp.uint16)
    u1 = lax.bitcast_convert_type(f1.astype(jnp.bfloat16), jnp.uint16)
    out_ref[0] = u0.astype(jnp.uint32) | (u1.astype(jnp.uint32) << 16)


def _pack_table(tv):
    out = pl.pallas_call(
        _pack_body,
        grid=(_N_LEVELS, 8),
        in_specs=[pl.BlockSpec((1, 512, 2, 128), lambda i, j: (i, j, 0, 0))],
        out_specs=pl.BlockSpec((1, 512, 128), lambda i, j: (i, j, 0)),
        out_shape=jax.ShapeDtypeStruct((_N_LEVELS, 4096, 128), jnp.uint32),
    )(tv)
    return out.reshape(_N_LEVELS * _T)


_MLP_B = 2048


def _mlp_body(enc_ref, w1_ref, w2_ref, out_ref):
    e = enc_ref[...]
    h = lax.dot_general(e, w1_ref[...], (((0,), (0,)), ((), ())),
                        preferred_element_type=jnp.float32)
    h = jnp.maximum(h, 0.0).reshape(_MLP_B, 64)
    o = jnp.dot(h, w2_ref[...], preferred_element_type=jnp.float32)
    out_ref[...] = jnp.clip(o[:, :3], 0.0, 1.0)


def _mlp(encT, W1, W2p):
    return pl.pallas_call(
        _mlp_body,
        grid=(_N // _MLP_B,),
        in_specs=[
            pl.BlockSpec((2 * _N_LEVELS, _MLP_B // 128, 128),
                         lambda i: (0, i, 0)),
            pl.BlockSpec((32, 64), lambda i: (0, 0)),
            pl.BlockSpec((64, 8), lambda i: (0, 0)),
        ],
        out_specs=pl.BlockSpec((_MLP_B, 3), lambda i: (i, 0)),
        out_shape=jax.ShapeDtypeStruct((_N, 3), jnp.float32),
    )(encT, W1, W2p)


def kernel(x, table, W1, W2):
    xf = x.reshape(_N * 3)
    tv = table.reshape(_N_LEVELS, _T // 128, 128, 2).transpose(0, 1, 3, 2)
    tab = _pack_table(tv)
    encT = _encode(xf, tab)
    W2p = jnp.zeros((64, 8), jnp.float32).at[:, :3].set(W2)
    return _mlp(encT, W1, W2p)


# speedup vs baseline: 3.4279x; 1.1506x over previous
import numpy as np
import jax
import jax.numpy as jnp
from jax import lax
from jax.experimental import pallas as pl
from jax.experimental.pallas import tpu as pltpu
from jax.experimental.pallas import tpu_sc as plsc

_N_LEVELS = 16
_LOG2_T = 19
_T = 1 << _LOG2_T
_BASE_RES = 16
_SCALE = 1.26
_N = 1048576
_PRIME1 = np.int32(np.uint32(2654435761).view(np.int32))
_PRIME2 = np.int32(805459861)
_RES = [int(np.ceil(_BASE_RES * (_SCALE ** l))) for l in range(_N_LEVELS)]
_DENSE = [(r + 1) ** 3 <= _T for r in _RES]

_NC, _NS, _L = 2, 16, 16
_NW = _NC * _NS
_PTS_PER_TILE = _N // _NW
_P = 2048
_CHUNKS = _PTS_PER_TILE // _P
_STRIPS = _P // _L
_NIDX = 8 * _P


def _encode_body(xf, tab, encT, xbuf, idxb0, idxb1, wcb0, wcb1, rf0, rf1,
                 fa00, fa01, fa10, fa11, semg0, semg1, sems0, sems1):
    wid = lax.axis_index("s") * _NC + lax.axis_index("c")
    tile_base = wid * _PTS_PER_TILE
    idxb = (idxb0, idxb1)
    wcb = (wcb0, wcb1)
    rf = (rf0, rf1)
    fa = ((fa00, fa01), (fa10, fa11))
    semg = (semg0, semg1)
    sems = (sems0, sems1)

    def _phase_a(l, idxr, wcr):
        res = _RES[l]
        dense = _DENSE[l]
        fres = jnp.float32(res)

        @pl.loop(0, _STRIPS)
        def _pa(s):
            o = s * _L
            xi = lax.iota(jnp.int32, _L) * 3 + (3 * o)
            xs = (plsc.load_gather(xbuf, [xi]),
                  plsc.load_gather(xbuf, [xi + 1]),
                  plsc.load_gather(xbuf, [xi + 2]))
            pis = []
            wlo = []
            whi = []
            for d in range(3):
                pos = xs[d] * fres
                pi = pos.astype(jnp.int32)
                fr = pos - pi.astype(jnp.float32)
                w = fr * fr * (3.0 - 2.0 * fr)
                pis.append(pi)
                whi.append(w)
                wlo.append(1.0 - w)
            for corner in range(8):
                bits = [(corner >> d) & 1 for d in range(3)]
                c0 = pis[0] + bits[0] if bits[0] else pis[0]
                c1 = pis[1] + bits[1] if bits[1] else pis[1]
                c2 = pis[2] + bits[2] if bits[2] else pis[2]
                if dense:
                    idx = c0 + c1 * (res + 1) + c2 * ((res + 1) * (res + 1))
                else:
                    idx = (c0 ^ (c1 * _PRIME1) ^ (c2 * _PRIME2)) & (_T - 1)
                wc = ((whi[0] if bits[0] else wlo[0])
                      * (whi[1] if bits[1] else wlo[1])
                      * (whi[2] if bits[2] else wlo[2]))
                idxr[pl.ds(corner * _P + o, _L)] = idx + (l * _T)
                wcr[pl.ds(corner * _P + o, _L)] = wc

    def _phase_c(wcr, rfr, f0a, f1a):
        @pl.loop(0, _P // 128)
        def _pcg(cg):
            @pl.loop(0, 128 // _L)
            def _pc(ss):
                o = cg * 128 + ss * _L
                acc0 = jnp.zeros((_L,), jnp.float32)
                acc1 = jnp.zeros((_L,), jnp.float32)
                for corner in range(8):
                    wc = wcr[pl.ds(corner * _P + o, _L)]
                    r = rfr[pl.ds(corner * _P + o, _L)]
                    ab = plsc.bitcast(r, jnp.bfloat16)
                    f0, f1 = plsc.unpack(ab, format=plsc.PackFormat.INTERLEAVED)
                    acc0 = acc0 + f0 * wc
                    acc1 = acc1 + f1 * wc
                f0a[cg, pl.ds(ss * _L, _L)] = acc0
                f1a[cg, pl.ds(ss * _L, _L)] = acc1

    @pl.loop(0, _CHUNKS)
    def _chunk(ci):
        base = tile_base + ci * _P
        cbase = wid * (_PTS_PER_TILE // 128) + ci * (_P // 128)
        pltpu.sync_copy(xf.at[pl.ds(3 * base, 3 * _P)], xbuf)

        def _scatter(l, p):
            return (
                pltpu.async_copy(
                    fa[p][0], encT.at[2 * l, pl.ds(cbase, _P // 128), :],
                    sems[p]),
                pltpu.async_copy(
                    fa[p][1], encT.at[2 * l + 1, pl.ds(cbase, _P // 128), :],
                    sems[p]),
            )

        gh = [None] * _N_LEVELS
        sh = [None, None]
        _phase_a(0, idxb[0], wcb[0])
        gh[0] = pltpu.async_copy(tab.at[idxb[0]], rf[0], semg[0])
        for l in range(1, _N_LEVELS):
            p, q = l % 2, (l - 1) % 2
            _phase_a(l, idxb[p], wcb[p])
            gh[l - 1].wait()
            gh[l] = pltpu.async_copy(tab.at[idxb[p]], rf[p], semg[p])
            if sh[q] is not None:
                sh[q][0].wait()
                sh[q][1].wait()
            _phase_c(wcb[q], rf[q], fa[q][0], fa[q][1])
            sh[q] = _scatter(l - 1, q)
        gh[_N_LEVELS - 1].wait()
        p = (_N_LEVELS - 1) % 2
        if sh[p] is not None:
            sh[p][0].wait()
            sh[p][1].wait()
        _phase_c(wcb[p], rf[p], fa[p][0], fa[p][1])
        sh[p] = _scatter(_N_LEVELS - 1, p)
        for pp in (0, 1):
            sh[pp][0].wait()
            sh[pp][1].wait()


def _encode(xT, tab):
    mesh = plsc.VectorSubcoreMesh(core_axis_name="c", subcore_axis_name="s")
    return pl.kernel(
        _encode_body,
        out_type=jax.ShapeDtypeStruct((2 * _N_LEVELS, _N // 128, 128),
                                      jnp.float32),
        mesh=mesh,
        compiler_params=pltpu.CompilerParams(
            use_tc_tiling_on_sc=False, needs_layout_passes=False),
        scratch_types=[
            pltpu.VMEM((3 * _P,), jnp.float32),
            pltpu.VMEM((_NIDX,), jnp.int32),
            pltpu.VMEM((_NIDX,), jnp.int32),
            pltpu.VMEM((_NIDX,), jnp.float32),
            pltpu.VMEM((_NIDX,), jnp.float32),
            pltpu.VMEM((_NIDX,), jnp.uint32),
            pltpu.VMEM((_NIDX,), jnp.uint32),
            pltpu.VMEM((_P // 128, 128), jnp.float32),
            pltpu.VMEM((_P // 128, 128), jnp.float32),
            pltpu.VMEM((_P // 128, 128), jnp.float32),
            pltpu.VMEM((_P // 128, 128), jnp.float32),
            pltpu.SemaphoreType.DMA,
            pltpu.SemaphoreType.DMA,
            pltpu.SemaphoreType.DMA,
            pltpu.SemaphoreType.DMA,
        ],
    )(xT, tab)


def _pack_body(tv_ref, out_ref):
    f0 = tv_ref[0, :, 0, :]
    f1 = tv_ref[0, :, 1, :]
    u0 = lax.bitcast_convert_type(f0.astype(jnp.bfloat16), jnp.uint16)
    u1 = lax.bitcast_convert_type(f1.astype(jnp.bfloat16), jnp.uint16)
    out_ref[0] = u0.astype(jnp.uint32) | (u1.astype(jnp.uint32) << 16)


def _pack_table(tv):
    out = pl.pallas_call(
        _pack_body,
        grid=(_N_LEVELS, 8),
        in_specs=[pl.BlockSpec((1, 512, 2, 128), lambda i, j: (i, j, 0, 0))],
        out_specs=pl.BlockSpec((1, 512, 128), lambda i, j: (i, j, 0)),
        out_shape=jax.ShapeDtypeStruct((_N_LEVELS, 4096, 128), jnp.uint32),
    )(tv)
    return out.reshape(_N_LEVELS * _T)


_MLP_B = 2048


def _mlp_body(enc_ref, w1_ref, w2_ref, out_ref):
    e = enc_ref[...]
    h = lax.dot_general(e, w1_ref[...], (((0,), (0,)), ((), ())),
                        preferred_element_type=jnp.float32)
    h = jnp.maximum(h, 0.0).reshape(_MLP_B, 64)
    o = jnp.dot(h, w2_ref[...], preferred_element_type=jnp.float32)
    out_ref[...] = jnp.clip(o[:, :3], 0.0, 1.0)


def _mlp(encT, W1, W2p):
    return pl.pallas_call(
        _mlp_body,
        grid=(_N // _MLP_B,),
        in_specs=[
            pl.BlockSpec((2 * _N_LEVELS, _MLP_B // 128, 128),
                         lambda i: (0, i, 0)),
            pl.BlockSpec((32, 64), lambda i: (0, 0)),
            pl.BlockSpec((64, 8), lambda i: (0, 0)),
        ],
        out_specs=pl.BlockSpec((_MLP_B, 3), lambda i: (i, 0)),
        out_shape=jax.ShapeDtypeStruct((_N, 3), jnp.float32),
    )(encT, W1, W2p)


def kernel(x, table, W1, W2):
    xf = x.reshape(_N * 3)
    tv = table.reshape(_N_LEVELS, _T // 128, 128, 2).transpose(0, 1, 3, 2)
    tab = _pack_table(tv)
    encT = _encode(xf, tab)
    W2p = jnp.zeros((64, 8), jnp.float32).at[:, :3].set(W2)
    return _mlp(encT, W1, W2p)


# speedup vs baseline: 6.8384x; 1.9949x over previous
import numpy as np
import jax
import jax.numpy as jnp
from jax import lax
from jax.experimental import pallas as pl
from jax.experimental.pallas import tpu as pltpu
from jax.experimental.pallas import tpu_sc as plsc

_N_LEVELS = 16
_LOG2_T = 19
_T = 1 << _LOG2_T
_BASE_RES = 16
_SCALE = 1.26
_N = 1048576
_PRIME1 = np.int32(np.uint32(2654435761).view(np.int32))
_PRIME2 = np.int32(805459861)
_RES = [int(np.ceil(_BASE_RES * (_SCALE ** l))) for l in range(_N_LEVELS)]
_DENSE = [(r + 1) ** 3 <= _T for r in _RES]
_N_DENSE = sum(_DENSE)

_NC, _NS, _L = 2, 16, 16
_NW = _NC * _NS
_PTS_PER_TILE = _N // _NW
_P = 1024
_CHUNKS = _PTS_PER_TILE // _P
_STRIPS = _P // _L
_NIDX = 8 * _P
_SL = _T // _NS


def _encode_body(xf, tab, lvlf, lvli, encT, spm, xbuf,
                 idxb0, idxb1, wcb0, wcb1, rf0, rf1,
                 fa00, fa01, fa10, fa11, lvlf_s, lvli_s,
                 semg0, semg1, sems0, sems1):
    sid = lax.axis_index("s")
    wid = sid * _NC + lax.axis_index("c")
    tile_base = wid * _PTS_PER_TILE
    idxb = (idxb0, idxb1)
    wcb = (wcb0, wcb1)
    rf = (rf0, rf1)
    fa = ((fa00, fa01), (fa10, fa11))
    semg = (semg0, semg1)
    sems = (sems0, sems1)

    pltpu.sync_copy(lvlf, lvlf_s)
    pltpu.sync_copy(lvli, lvli_s)

    def _phase_a(dense, fres, s1, s2, ci, idxr, wcr):
        base = tile_base + ci * _P
        pltpu.sync_copy(xf.at[pl.ds(3 * base, 3 * _P)], xbuf)

        @pl.loop(0, _STRIPS)
        def _pa(s):
            o = s * _L
            xi = lax.iota(jnp.int32, _L) * 3 + (3 * o)
            xs = (plsc.load_gather(xbuf, [xi]),
                  plsc.load_gather(xbuf, [xi + 1]),
                  plsc.load_gather(xbuf, [xi + 2]))
            pis = []
            wlo = []
            whi = []
            for d in range(3):
                pos = xs[d] * fres
                pi = pos.astype(jnp.int32)
                fr = pos - pi.astype(jnp.float32)
                w = fr * fr * (3.0 - 2.0 * fr)
                pis.append(pi)
                whi.append(w)
                wlo.append(1.0 - w)
            for corner in range(8):
                bits = [(corner >> d) & 1 for d in range(3)]
                c0 = pis[0] + bits[0] if bits[0] else pis[0]
                c1 = pis[1] + bits[1] if bits[1] else pis[1]
                c2 = pis[2] + bits[2] if bits[2] else pis[2]
                if dense:
                    idx = c0 + c1 * s1 + c2 * s2
                else:
                    idx = (c0 ^ (c1 * _PRIME1) ^ (c2 * _PRIME2)) & (_T - 1)
                wc = ((whi[0] if bits[0] else wlo[0])
                      * (whi[1] if bits[1] else wlo[1])
                      * (whi[2] if bits[2] else wlo[2]))
                idxr[pl.ds(corner * _P + o, _L)] = idx
                wcr[pl.ds(corner * _P + o, _L)] = wc

    def _phase_c(wcr, rfr, f0a, f1a):
        @pl.loop(0, _P // 128)
        def _pcg(cg):
            @pl.loop(0, 128 // _L)
            def _pc(ss):
                o = cg * 128 + ss * _L
                acc0 = jnp.zeros((_L,), jnp.float32)
                acc1 = jnp.zeros((_L,), jnp.float32)
                for corner in range(8):
                    wc = wcr[pl.ds(corner * _P + o, _L)]
                    r = rfr[pl.ds(corner * _P + o, _L)]
                    ab = plsc.bitcast(r, jnp.bfloat16)
                    f0, f1 = plsc.unpack(ab, format=plsc.PackFormat.INTERLEAVED)
                    acc0 = acc0 + f0 * wc
                    acc1 = acc1 + f1 * wc
                f0a[cg, pl.ds(ss * _L, _L)] = acc0
                f1a[cg, pl.ds(ss * _L, _L)] = acc1

    def _scatter(l, ci, p):
        cbase = wid * (_PTS_PER_TILE // 128) + ci * (_P // 128)
        pltpu.async_copy(fa[p][0], encT.at[2 * l, pl.ds(cbase, _P // 128), :],
                         sems[p])
        pltpu.async_copy(fa[p][1],
                         encT.at[2 * l + 1, pl.ds(cbase, _P // 128), :],
                         sems[p])

    def _gather(p):
        pltpu.async_copy(spm.at[idxb[p]], rf[p], semg[p])

    def _gwait(p):
        pltpu.make_async_copy(spm.at[idxb[p]], rf[p], semg[p]).wait()

    def _swait(p):
        pltpu.make_async_copy(fa[p][0], encT.at[0, pl.ds(0, _P // 128), :],
                              sems[p]).wait()
        pltpu.make_async_copy(fa[p][1], encT.at[0, pl.ds(0, _P // 128), :],
                              sems[p]).wait()

    def _run_levels(lo, hi, dense):
        @pl.loop(lo, hi)
        def _lvl(l):
            lv = jnp.zeros((_L,), jnp.int32) + l
            fres = plsc.load_gather(lvlf_s, [lv])
            s1 = plsc.load_gather(lvli_s, [2 * lv])
            s2 = plsc.load_gather(lvli_s, [2 * lv + 1])

            pltpu.sync_copy(tab.at[pl.ds(l * _T + sid * _SL, _SL)],
                            spm.at[pl.ds(sid * _SL, _SL)])
            plsc.subcore_barrier()

            _phase_a(dense, fres, s1, s2, 0, idxb[0], wcb[0])
            _gather(0)

            @pl.loop(0, _CHUNKS // 2)
            def _ck(k):
                c0 = 2 * k
                _phase_a(dense, fres, s1, s2, c0 + 1, idxb[1], wcb[1])
                _gwait(0)
                _gather(1)

                @pl.when(k > 0)
                def _():
                    _swait(0)
                _phase_c(wcb[0], rf[0], fa[0][0], fa[0][1])
                _scatter(l, c0, 0)

                @pl.when(k < _CHUNKS // 2 - 1)
                def _():
                    _phase_a(dense, fres, s1, s2, c0 + 2, idxb[0], wcb[0])
                _gwait(1)

                @pl.when(k < _CHUNKS // 2 - 1)
                def _():
                    _gather(0)

                @pl.when(k > 0)
                def _():
                    _swait(1)
                _phase_c(wcb[1], rf[1], fa[1][0], fa[1][1])
                _scatter(l, c0 + 1, 1)

            for p in (0, 1):
                _swait(p)
            plsc.subcore_barrier()

    _run_levels(0, _N_DENSE, True)
    _run_levels(_N_DENSE, _N_LEVELS, False)


def _encode(xf, tab, lvlf, lvli):
    mesh = plsc.VectorSubcoreMesh(core_axis_name="c", subcore_axis_name="s")
    return pl.kernel(
        _encode_body,
        out_type=jax.ShapeDtypeStruct((2 * _N_LEVELS, _N // 128, 128),
                                      jnp.float32),
        mesh=mesh,
        compiler_params=pltpu.CompilerParams(
            use_tc_tiling_on_sc=False, needs_layout_passes=False),
        scratch_types=[
            pltpu.VMEM_SHARED((_T,), jnp.uint32),
            pltpu.VMEM((3 * _P,), jnp.float32),
            pltpu.VMEM((_NIDX,), jnp.int32),
            pltpu.VMEM((_NIDX,), jnp.int32),
            pltpu.VMEM((_NIDX,), jnp.float32),
            pltpu.VMEM((_NIDX,), jnp.float32),
            pltpu.VMEM((_NIDX,), jnp.uint32),
            pltpu.VMEM((_NIDX,), jnp.uint32),
            pltpu.VMEM((_P // 128, 128), jnp.float32),
            pltpu.VMEM((_P // 128, 128), jnp.float32),
            pltpu.VMEM((_P // 128, 128), jnp.float32),
            pltpu.VMEM((_P // 128, 128), jnp.float32),
            pltpu.VMEM((_N_LEVELS,), jnp.float32),
            pltpu.VMEM((2 * _N_LEVELS,), jnp.int32),
            pltpu.SemaphoreType.DMA,
            pltpu.SemaphoreType.DMA,
            pltpu.SemaphoreType.DMA,
            pltpu.SemaphoreType.DMA,
        ],
    )(xf, tab, lvlf, lvli)


def _pack_body(tv_ref, out_ref):
    f0 = tv_ref[0, :, 0, :]
    f1 = tv_ref[0, :, 1, :]
    u0 = lax.bitcast_convert_type(f0.astype(jnp.bfloat16), jnp.uint16)
    u1 = lax.bitcast_convert_type(f1.astype(jnp.bfloat16), jnp.uint16)
    out_ref[0] = u0.astype(jnp.uint32) | (u1.astype(jnp.uint32) << 16)


def _pack_table(tv):
    out = pl.pallas_call(
        _pack_body,
        grid=(_N_LEVELS, 8),
        in_specs=[pl.BlockSpec((1, 512, 2, 128), lambda i, j: (i, j, 0, 0))],
        out_specs=pl.BlockSpec((1, 512, 128), lambda i, j: (i, j, 0)),
        out_shape=jax.ShapeDtypeStruct((_N_LEVELS, 4096, 128), jnp.uint32),
    )(tv)
    return out.reshape(_N_LEVELS * _T)


_MLP_B = 2048


def _mlp_body(enc_ref, w1_ref, w2_ref, out_ref):
    e = enc_ref[...]
    h = lax.dot_general(e, w1_ref[...], (((0,), (0,)), ((), ())),
                        preferred_element_type=jnp.float32)
    h = jnp.maximum(h, 0.0).reshape(_MLP_B, 64)
    o = jnp.dot(h, w2_ref[...], preferred_element_type=jnp.float32)
    out_ref[...] = jnp.clip(o[:, :3], 0.0, 1.0)


def _mlp(encT, W1, W2p):
    return pl.pallas_call(
        _mlp_body,
        grid=(_N // _MLP_B,),
        in_specs=[
            pl.BlockSpec((2 * _N_LEVELS, _MLP_B // 128, 128),
                         lambda i: (0, i, 0)),
            pl.BlockSpec((32, 64), lambda i: (0, 0)),
            pl.BlockSpec((64, 8), lambda i: (0, 0)),
        ],
        out_specs=pl.BlockSpec((_MLP_B, 3), lambda i: (i, 0)),
        out_shape=jax.ShapeDtypeStruct((_N, 3), jnp.float32),
    )(encT, W1, W2p)


def kernel(x, table, W1, W2):
    xf = x.reshape(_N * 3)
    tv = table.reshape(_N_LEVELS, _T // 128, 128, 2).transpose(0, 1, 3, 2)
    tab = _pack_table(tv)
    lvlf = jnp.asarray(np.array(_RES, dtype=np.float32))
    strides = []
    for r in _RES:
        strides += [r + 1, (r + 1) * (r + 1)]
    lvli = jnp.asarray(np.array(strides, dtype=np.int32))
    encT = _encode(xf, tab, lvlf, lvli)
    W2p = jnp.zeros((64, 8), jnp.float32).at[:, :3].set(W2)
    return _mlp(encT, W1, W2p)


# speedup vs baseline: 8.8720x; 1.2974x over previous
import numpy as np
import jax
import jax.numpy as jnp
from jax import lax
from jax.experimental import pallas as pl
from jax.experimental.pallas import tpu as pltpu
from jax.experimental.pallas import tpu_sc as plsc

_N_LEVELS = 16
_LOG2_T = 19
_T = 1 << _LOG2_T
_BASE_RES = 16
_SCALE = 1.26
_N = 1048576
_PRIME1 = np.int32(np.uint32(2654435761).view(np.int32))
_PRIME2 = np.int32(805459861)
_RES = [int(np.ceil(_BASE_RES * (_SCALE ** l))) for l in range(_N_LEVELS)]
_DENSE = [(r + 1) ** 3 <= _T for r in _RES]
_N_DENSE = sum(_DENSE)

_NC, _NS, _L = 2, 16, 16
_NW = _NC * _NS
_PTS_PER_TILE = _N // _NW
_P = 1024
_CHUNKS = _PTS_PER_TILE // _P
_STRIPS = _P // _L
_NIDX = 8 * _P
_SL = _T // _NS


def _encode_body(xf, tab, lvlf, lvli, encT, spm, xb0, xb1, xb2,
                 idxb0, idxb1, wcb0, wcb1, rf0, rf1,
                 fa00, fa01, fa10, fa11, lvlf_s, lvli_s,
                 semg0, semg1, sems0, sems1):
    sid = lax.axis_index("s")
    wid = sid * _NC + lax.axis_index("c")
    tile_base = wid * _PTS_PER_TILE
    idxb = (idxb0, idxb1)
    wcb = (wcb0, wcb1)
    rf = (rf0, rf1)
    fa = ((fa00, fa01), (fa10, fa11))
    semg = (semg0, semg1)
    sems = (sems0, sems1)

    pltpu.sync_copy(lvlf, lvlf_s)
    pltpu.sync_copy(lvli, lvli_s)

    def _phase_a(dense, fres, s1, s2, ci, idxr, wcr):
        cb = wid * (_PTS_PER_TILE // 128) + ci * (_P // 128)
        pltpu.sync_copy(xf.at[0, pl.ds(cb, _P // 128), :], xb0)
        pltpu.sync_copy(xf.at[1, pl.ds(cb, _P // 128), :], xb1)
        pltpu.sync_copy(xf.at[2, pl.ds(cb, _P // 128), :], xb2)

        @pl.loop(0, _P // 128)
        def _pag(cg):
            @pl.loop(0, 128 // _L)
            def _pa(ss):
                o = cg * 128 + ss * _L
                xs = (xb0[cg, pl.ds(ss * _L, _L)],
                      xb1[cg, pl.ds(ss * _L, _L)],
                      xb2[cg, pl.ds(ss * _L, _L)])
                pis = []
                wlo = []
                whi = []
                for d in range(3):
                    pos = xs[d] * fres
                    pi = pos.astype(jnp.int32)
                    fr = pos - pi.astype(jnp.float32)
                    w = fr * fr * (3.0 - 2.0 * fr)
                    pis.append(pi)
                    whi.append(w)
                    wlo.append(1.0 - w)
                for corner in range(8):
                    bits = [(corner >> d) & 1 for d in range(3)]
                    c0 = pis[0] + bits[0] if bits[0] else pis[0]
                    c1 = pis[1] + bits[1] if bits[1] else pis[1]
                    c2 = pis[2] + bits[2] if bits[2] else pis[2]
                    if dense:
                        idx = c0 + c1 * s1 + c2 * s2
                    else:
                        idx = (c0 ^ (c1 * _PRIME1) ^ (c2 * _PRIME2)) & (_T - 1)
                    wc = ((whi[0] if bits[0] else wlo[0])
                          * (whi[1] if bits[1] else wlo[1])
                          * (whi[2] if bits[2] else wlo[2]))
                    idxr[pl.ds(corner * _P + o, _L)] = idx
                    wcr[pl.ds(corner * _P + o, _L)] = wc

    def _phase_c(wcr, rfr, f0a, f1a):
        @pl.loop(0, _P // 128)
        def _pcg(cg):
            @pl.loop(0, 128 // _L)
            def _pc(ss):
                o = cg * 128 + ss * _L
                acc0 = jnp.zeros((_L,), jnp.float32)
                acc1 = jnp.zeros((_L,), jnp.float32)
                for corner in range(8):
                    wc = wcr[pl.ds(corner * _P + o, _L)]
                    r = rfr[pl.ds(corner * _P + o, _L)]
                    ab = plsc.bitcast(r, jnp.bfloat16)
                    f0, f1 = plsc.unpack(ab, format=plsc.PackFormat.INTERLEAVED)
                    acc0 = acc0 + f0 * wc
                    acc1 = acc1 + f1 * wc
                f0a[cg, pl.ds(ss * _L, _L)] = acc0
                f1a[cg, pl.ds(ss * _L, _L)] = acc1

    def _scatter(l, ci, p):
        cbase = wid * (_PTS_PER_TILE // 128) + ci * (_P // 128)
        pltpu.async_copy(fa[p][0], encT.at[2 * l, pl.ds(cbase, _P // 128), :],
                         sems[p])
        pltpu.async_copy(fa[p][1],
                         encT.at[2 * l + 1, pl.ds(cbase, _P // 128), :],
                         sems[p])

    def _gather(p):
        pltpu.async_copy(spm.at[idxb[p]], rf[p], semg[p])

    def _gwait(p):
        pltpu.make_async_copy(spm.at[idxb[p]], rf[p], semg[p]).wait()

    def _swait(p):
        pltpu.make_async_copy(fa[p][0], encT.at[0, pl.ds(0, _P // 128), :],
                              sems[p]).wait()
        pltpu.make_async_copy(fa[p][1], encT.at[0, pl.ds(0, _P // 128), :],
                              sems[p]).wait()

    def _run_levels(lo, hi, dense):
        @pl.loop(lo, hi)
        def _lvl(l):
            lv = jnp.zeros((_L,), jnp.int32) + l
            fres = plsc.load_gather(lvlf_s, [lv])
            s1 = plsc.load_gather(lvli_s, [2 * lv])
            s2 = plsc.load_gather(lvli_s, [2 * lv + 1])

            pltpu.sync_copy(tab.at[pl.ds(l * _T + sid * _SL, _SL)],
                            spm.at[pl.ds(sid * _SL, _SL)])
            plsc.subcore_barrier()

            _phase_a(dense, fres, s1, s2, 0, idxb[0], wcb[0])
            _gather(0)

            @pl.loop(0, _CHUNKS // 2)
            def _ck(k):
                c0 = 2 * k
                _phase_a(dense, fres, s1, s2, c0 + 1, idxb[1], wcb[1])
                _gwait(0)
                _gather(1)

                @pl.when(k > 0)
                def _():
                    _swait(0)
                _phase_c(wcb[0], rf[0], fa[0][0], fa[0][1])
                _scatter(l, c0, 0)

                @pl.when(k < _CHUNKS // 2 - 1)
                def _():
                    _phase_a(dense, fres, s1, s2, c0 + 2, idxb[0], wcb[0])
                _gwait(1)

                @pl.when(k < _CHUNKS // 2 - 1)
                def _():
                    _gather(0)

                @pl.when(k > 0)
                def _():
                    _swait(1)
                _phase_c(wcb[1], rf[1], fa[1][0], fa[1][1])
                _scatter(l, c0 + 1, 1)

            for p in (0, 1):
                _swait(p)
            plsc.subcore_barrier()

    _run_levels(0, _N_DENSE, True)
    _run_levels(_N_DENSE, _N_LEVELS, False)


def _encode(xf, tab, lvlf, lvli):
    mesh = plsc.VectorSubcoreMesh(core_axis_name="c", subcore_axis_name="s")
    return pl.kernel(
        _encode_body,
        out_type=jax.ShapeDtypeStruct((2 * _N_LEVELS, _N // 128, 128),
                                      jnp.float32),
        mesh=mesh,
        compiler_params=pltpu.CompilerParams(
            use_tc_tiling_on_sc=False, needs_layout_passes=False),
        scratch_types=[
            pltpu.VMEM_SHARED((_T,), jnp.uint32),
            pltpu.VMEM((_P // 128, 128), jnp.float32),
            pltpu.VMEM((_P // 128, 128), jnp.float32),
            pltpu.VMEM((_P // 128, 128), jnp.float32),
            pltpu.VMEM((_NIDX,), jnp.int32),
            pltpu.VMEM((_NIDX,), jnp.int32),
            pltpu.VMEM((_NIDX,), jnp.float32),
            pltpu.VMEM((_NIDX,), jnp.float32),
            pltpu.VMEM((_NIDX,), jnp.uint32),
            pltpu.VMEM((_NIDX,), jnp.uint32),
            pltpu.VMEM((_P // 128, 128), jnp.float32),
            pltpu.VMEM((_P // 128, 128), jnp.float32),
            pltpu.VMEM((_P // 128, 128), jnp.float32),
            pltpu.VMEM((_P // 128, 128), jnp.float32),
            pltpu.VMEM((_N_LEVELS,), jnp.float32),
            pltpu.VMEM((2 * _N_LEVELS,), jnp.int32),
            pltpu.SemaphoreType.DMA,
            pltpu.SemaphoreType.DMA,
            pltpu.SemaphoreType.DMA,
            pltpu.SemaphoreType.DMA,
        ],
    )(xf, tab, lvlf, lvli)


def _pack_body(tv_ref, out_ref):
    f0 = tv_ref[0, :, 0, :]
    f1 = tv_ref[0, :, 1, :]
    u0 = lax.bitcast_convert_type(f0.astype(jnp.bfloat16), jnp.uint16)
    u1 = lax.bitcast_convert_type(f1.astype(jnp.bfloat16), jnp.uint16)
    out_ref[0] = u0.astype(jnp.uint32) | (u1.astype(jnp.uint32) << 16)


def _pack_table(tv):
    out = pl.pallas_call(
        _pack_body,
        grid=(_N_LEVELS, 8),
        in_specs=[pl.BlockSpec((1, 512, 2, 128), lambda i, j: (i, j, 0, 0))],
        out_specs=pl.BlockSpec((1, 512, 128), lambda i, j: (i, j, 0)),
        out_shape=jax.ShapeDtypeStruct((_N_LEVELS, 4096, 128), jnp.uint32),
    )(tv)
    return out.reshape(_N_LEVELS * _T)


_XB = 8192


def _xfmt_body(xv_ref, out_ref):
    out_ref[...] = xv_ref[...].reshape(3, _XB // 128, 128)


def _xfmt(xv):
    return pl.pallas_call(
        _xfmt_body,
        grid=(_N // _XB,),
        in_specs=[pl.BlockSpec((3, _XB), lambda i: (0, i))],
        out_specs=pl.BlockSpec((3, _XB // 128, 128), lambda i: (0, i, 0)),
        out_shape=jax.ShapeDtypeStruct((3, _N // 128, 128), jnp.float32),
    )(xv)


_MLP_B = 8192


def _mlp_body(enc_ref, w1_ref, w2_ref, out_ref):
    e = enc_ref[...]
    h = lax.dot_general(e, w1_ref[...], (((0,), (0,)), ((), ())),
                        preferred_element_type=jnp.float32)
    h = jnp.maximum(h, 0.0).reshape(_MLP_B, 64)
    o = jnp.dot(h, w2_ref[...], preferred_element_type=jnp.float32)
    out_ref[...] = jnp.clip(o[:, :3], 0.0, 1.0)


def _mlp(encT, W1, W2p):
    return pl.pallas_call(
        _mlp_body,
        grid=(_N // _MLP_B,),
        in_specs=[
            pl.BlockSpec((2 * _N_LEVELS, _MLP_B // 128, 128),
                         lambda i: (0, i, 0)),
            pl.BlockSpec((32, 64), lambda i: (0, 0)),
            pl.BlockSpec((64, 8), lambda i: (0, 0)),
        ],
        out_specs=pl.BlockSpec((_MLP_B, 3), lambda i: (i, 0)),
        out_shape=jax.ShapeDtypeStruct((_N, 3), jnp.float32),
    )(encT, W1, W2p)


def kernel(x, table, W1, W2):
    xf = _xfmt(x.T)
    tv = table.reshape(_N_LEVELS, _T // 128, 128, 2).transpose(0, 1, 3, 2)
    tab = _pack_table(tv)
    lvlf = jnp.asarray(np.array(_RES, dtype=np.float32))
    strides = []
    for r in _RES:
        strides += [r + 1, (r + 1) * (r + 1)]
    lvli = jnp.asarray(np.array(strides, dtype=np.int32))
    encT = _encode(xf, tab, lvlf, lvli)
    W2p = jnp.zeros((64, 8), jnp.float32).at[:, :3].set(W2)
    return _mlp(encT, W1, W2p)
